# baseline (speedup 1.0000x reference)
import functools

import jax
import jax.numpy as jnp
from jax import lax
from jax.experimental import pallas as pl
from jax.experimental.pallas import tpu as pltpu
from jax.experimental.pallas import tpu_sc as plsc

VOCAB = 100000
POS = 1024
DIM = 128
B = 4096
S = 200
N = B * S
NC = 2
NS = 16
NW = NC * NS
RPW = N // NW
SUB = 128
NCH = RPW // SUB
LANES = 16
NSLOT = 5


def _combo_body(pad_ref, pos_ref, out_ref):
    out_ref[...] = pad_ref[...][:, None, :] + pos_ref[...][None, :, :]


def _make_combo(padding_embedding, pos_embedding):
    combo3 = pl.pallas_call(
        _combo_body,
        out_shape=jax.ShapeDtypeStruct((3, S, DIM), jnp.float32),
    )(padding_embedding, pos_embedding[:S])
    return combo3.reshape(3 * S, DIM)


def _sc_body(char_hbm, pad_hbm, emb_hbm, combo_hbm, out_hbm,
             idx_e, idx_c, rows_e, combo_sh, *sems):
    assert len(sems) == 5 * NSLOT
    sem_ie = sems[0 * NSLOT:1 * NSLOT]
    sem_ic = sems[1 * NSLOT:2 * NSLOT]
    sem_ge = sems[2 * NSLOT:3 * NSLOT]
    sem_gc = sems[3 * NSLOT:4 * NSLOT]
    sem_s = sems[4 * NSLOT:5 * NSLOT]

    wid = lax.axis_index("s") * NC + lax.axis_index("c")
    base = wid * RPW
    lane = lax.iota(jnp.int32, LANES)

    def fire_idx(i, s):
        off = base + i * SUB
        pltpu.async_copy(char_hbm.at[pl.ds(off, SUB)], idx_e.at[s], sem_ie[s])
        pltpu.async_copy(pad_hbm.at[pl.ds(off, SUB)], idx_c.at[s], sem_ic[s])

    def wait_idx(i, s):
        off = base + i * SUB
        pltpu.make_async_copy(char_hbm.at[pl.ds(off, SUB)], idx_e.at[s],
                              sem_ie[s]).wait()
        pltpu.make_async_copy(pad_hbm.at[pl.ds(off, SUB)], idx_c.at[s],
                              sem_ic[s]).wait()

    def fix_idx(i, s):
        off = base + i * SUB

        def fix(k, c):
            sl = pl.ds(k * LANES, LANES)
            s_v = lax.rem(off + k * LANES + lane, S)
            idx_c[s, sl] = idx_c[s, sl] * S + s_v
            return c

        lax.fori_loop(0, SUB // LANES, fix, 0)

    def fire_ge(s):
        pltpu.async_copy(emb_hbm.at[idx_e.at[s]], rows_e.at[s], sem_ge[s])

    def wait_ge(s):
        pltpu.make_async_copy(emb_hbm.at[idx_e.at[s]], rows_e.at[s],
                              sem_ge[s]).wait()

    def fire_gc(s):
        pltpu.async_copy(combo_sh.at[idx_c.at[s]], rows_e.at[s], sem_gc[s],
                         add=True)

    def wait_gc(s):
        pltpu.make_async_copy(combo_sh.at[idx_c.at[s]], rows_e.at[s],
                              sem_gc[s]).wait()

    def fire_scatter(i, s):
        off = base + i * SUB
        pltpu.async_copy(rows_e.at[s], out_hbm.at[pl.ds(off, SUB)], sem_s[s])

    def wait_scatter(i, s):
        off = base + i * SUB
        pltpu.make_async_copy(rows_e.at[s], out_hbm.at[pl.ds(off, SUB)],
                              sem_s[s]).wait()

    stage_rows = 40
    stage_start = jnp.minimum(lax.axis_index("s") * stage_rows,
                              3 * S - stage_rows)
    pltpu.sync_copy(combo_hbm.at[pl.ds(stage_start, stage_rows)],
                    combo_sh.at[pl.ds(stage_start, stage_rows)])

    plsc.subcore_barrier()

    fire_idx(0, 0)
    fire_idx(1, 1)
    wait_idx(0, 0)
    fix_idx(0, 0)
    fire_ge(0)

    def body(t, carry):
        for j in range(NSLOT):
            i = NSLOT * t + j
            s_0 = j
            s_c = (j - 1) % NSLOT
            s_g = (j + 1) % NSLOT
            s_i = (j + 2) % NSLOT

            @pl.when(jnp.logical_and(i >= 1, i <= NCH))
            def _():
                wait_gc(s_c)
                fire_scatter(i - 1, s_c)

            @pl.when(jnp.logical_and(i >= NSLOT - 1, i <= NCH + NSLOT - 2))
            def _():
                wait_scatter(i + 1 - NSLOT, s_g)

            @pl.when(i <= NCH - 2)
            def _():
                wait_idx(i + 1, s_g)
                fix_idx(i + 1, s_g)
                fire_ge(s_g)

            @pl.when(i <= NCH - 1)
            def _():
                wait_ge(s_0)
                fire_gc(s_0)

            @pl.when(i <= NCH - 3)
            def _():
                fire_idx(i + 2, s_i)

        return carry

    n_iter = (NCH + NSLOT + NSLOT - 1) // NSLOT
    lax.fori_loop(0, n_iter, body, 0)


_sc_lookup = functools.partial(
    pl.kernel,
    mesh=plsc.VectorSubcoreMesh(core_axis_name="c", subcore_axis_name="s"),
    out_type=jax.ShapeDtypeStruct((N, DIM), jnp.float32),
    scratch_types=[
        pltpu.VMEM((NSLOT, SUB), jnp.int32),
        pltpu.VMEM((NSLOT, SUB), jnp.int32),
        pltpu.VMEM((NSLOT, SUB, DIM), jnp.float32),
        pltpu.VMEM_SHARED((3 * S, DIM), jnp.float32),
    ] + [pltpu.SemaphoreType.DMA] * (5 * NSLOT),
)(_sc_body)


@jax.jit
def kernel(char_ids, pad_ids, embedding, pos_embedding, padding_embedding):
    combo = _make_combo(padding_embedding, pos_embedding)
    char_flat = char_ids.reshape(N).astype(jnp.int32)
    pad_flat = pad_ids.reshape(N).astype(jnp.int32)
    out = _sc_lookup(char_flat, pad_flat, embedding, combo)
    return out.reshape(B, S, DIM)


# speedup vs baseline: 1.0026x; 1.0026x over previous
import functools

import jax
import jax.numpy as jnp
from jax import lax
from jax.experimental import pallas as pl
from jax.experimental.pallas import tpu as pltpu
from jax.experimental.pallas import tpu_sc as plsc

VOCAB = 100000
POS = 1024
DIM = 128
B = 4096
S = 200
N = B * S
NC = 2
NS = 16
NW = NC * NS
RPW = N // NW
SUB = 128
NCH = RPW // SUB
LANES = 16
NSLOT = 5


def _sc_body(char_hbm, pad_hbm, emb_hbm, pos_hbm, padtab_hbm, out_hbm,
             idx_e, idx_c, rows_e, pos_v, pad_v, combo_sh, *sems):
    assert len(sems) == 5 * NSLOT
    sem_ie = sems[0 * NSLOT:1 * NSLOT]
    sem_ic = sems[1 * NSLOT:2 * NSLOT]
    sem_ge = sems[2 * NSLOT:3 * NSLOT]
    sem_gc = sems[3 * NSLOT:4 * NSLOT]
    sem_s = sems[4 * NSLOT:5 * NSLOT]

    wid = lax.axis_index("s") * NC + lax.axis_index("c")
    base = wid * RPW
    lane = lax.iota(jnp.int32, LANES)

    def fire_idx(i, s):
        off = base + i * SUB
        pltpu.async_copy(char_hbm.at[pl.ds(off, SUB)], idx_e.at[s], sem_ie[s])
        pltpu.async_copy(pad_hbm.at[pl.ds(off, SUB)], idx_c.at[s], sem_ic[s])

    def wait_idx(i, s):
        off = base + i * SUB
        pltpu.make_async_copy(char_hbm.at[pl.ds(off, SUB)], idx_e.at[s],
                              sem_ie[s]).wait()
        pltpu.make_async_copy(pad_hbm.at[pl.ds(off, SUB)], idx_c.at[s],
                              sem_ic[s]).wait()

    def fix_idx(i, s):
        off = base + i * SUB

        def fix(k, c):
            sl = pl.ds(k * LANES, LANES)
            s_v = lax.rem(off + k * LANES + lane, S)
            idx_c[s, sl] = idx_c[s, sl] * S + s_v
            return c

        lax.fori_loop(0, SUB // LANES, fix, 0)

    def fire_ge(s):
        pltpu.async_copy(emb_hbm.at[idx_e.at[s]], rows_e.at[s], sem_ge[s])

    def wait_ge(s):
        pltpu.make_async_copy(emb_hbm.at[idx_e.at[s]], rows_e.at[s],
                              sem_ge[s]).wait()

    def fire_gc(s):
        pltpu.async_copy(combo_sh.at[idx_c.at[s]], rows_e.at[s], sem_gc[s],
                         add=True)

    def wait_gc(s):
        pltpu.make_async_copy(combo_sh.at[idx_c.at[s]], rows_e.at[s],
                              sem_gc[s]).wait()

    def fire_scatter(i, s):
        off = base + i * SUB
        pltpu.async_copy(rows_e.at[s], out_hbm.at[pl.ds(off, SUB)], sem_s[s])

    def wait_scatter(i, s):
        off = base + i * SUB
        pltpu.make_async_copy(rows_e.at[s], out_hbm.at[pl.ds(off, SUB)],
                              sem_s[s]).wait()

    stage_rows = 40
    stage_start = lax.axis_index("s") * stage_rows
    stage_start = jnp.minimum(stage_start, 3 * S - stage_rows)
    p0 = stage_start // S
    s0 = stage_start - p0 * S
    pltpu.sync_copy(pos_hbm.at[pl.ds(s0, stage_rows)], pos_v)
    pltpu.sync_copy(padtab_hbm, pad_v)

    def combo_row(r, c):
        for d in range(DIM // LANES):
            sl = pl.ds(d * LANES, LANES)
            rows_e[0, r, sl] = pos_v[r, sl] + pad_v[p0, sl]
        return c

    lax.fori_loop(0, stage_rows, combo_row, 0)
    pltpu.sync_copy(rows_e.at[0, pl.ds(0, stage_rows)],
                    combo_sh.at[pl.ds(stage_start, stage_rows)])

    plsc.subcore_barrier()

    fire_idx(0, 0)
    fire_idx(1, 1)
    wait_idx(0, 0)
    fix_idx(0, 0)
    fire_ge(0)

    def body(t, carry):
        for j in range(NSLOT):
            i = NSLOT * t + j
            s_0 = j
            s_c = (j - 1) % NSLOT
            s_g = (j + 1) % NSLOT
            s_i = (j + 2) % NSLOT

            @pl.when(jnp.logical_and(i >= 1, i <= NCH))
            def _():
                wait_gc(s_c)
                fire_scatter(i - 1, s_c)

            @pl.when(jnp.logical_and(i >= NSLOT - 1, i <= NCH + NSLOT - 2))
            def _():
                wait_scatter(i + 1 - NSLOT, s_g)

            @pl.when(i <= NCH - 2)
            def _():
                wait_idx(i + 1, s_g)
                fix_idx(i + 1, s_g)
                fire_ge(s_g)

            @pl.when(i <= NCH - 1)
            def _():
                wait_ge(s_0)
                fire_gc(s_0)

            @pl.when(i <= NCH - 3)
            def _():
                fire_idx(i + 2, s_i)

        return carry

    n_iter = (NCH + NSLOT + NSLOT - 1) // NSLOT
    lax.fori_loop(0, n_iter, body, 0)


_sc_lookup = functools.partial(
    pl.kernel,
    mesh=plsc.VectorSubcoreMesh(core_axis_name="c", subcore_axis_name="s"),
    out_type=jax.ShapeDtypeStruct((N, DIM), jnp.float32),
    scratch_types=[
        pltpu.VMEM((NSLOT, SUB), jnp.int32),
        pltpu.VMEM((NSLOT, SUB), jnp.int32),
        pltpu.VMEM((NSLOT, SUB, DIM), jnp.float32),
        pltpu.VMEM((40, DIM), jnp.float32),
        pltpu.VMEM((3, DIM), jnp.float32),
        pltpu.VMEM_SHARED((3 * S, DIM), jnp.float32),
    ] + [pltpu.SemaphoreType.DMA] * (5 * NSLOT),
)(_sc_body)


@jax.jit
def kernel(char_ids, pad_ids, embedding, pos_embedding, padding_embedding):
    char_flat = char_ids.reshape(N).astype(jnp.int32)
    pad_flat = pad_ids.reshape(N).astype(jnp.int32)
    out = _sc_lookup(char_flat, pad_flat, embedding, pos_embedding,
                     padding_embedding)
    return out.reshape(B, S, DIM)


# speedup vs baseline: 1.0091x; 1.0065x over previous
import functools

import jax
import jax.numpy as jnp
from jax import lax
from jax.experimental import pallas as pl
from jax.experimental.pallas import tpu as pltpu
from jax.experimental.pallas import tpu_sc as plsc

VOCAB = 100000
POS = 1024
DIM = 128
B = 4096
S = 200
N = B * S
NC = 2
NS = 16
NW = NC * NS
RPW = N // NW
SUB = 128
NCH = RPW // SUB
LANES = 16
NSLOT = 4


def _sc_body(char_hbm, pad_hbm, emb_hbm, pos_hbm, padtab_hbm, out_hbm,
             idx_e, idx_c, rows_e, pos_v, pad_v, combo_sh, *sems):
    assert len(sems) == 2 + 3 * NSLOT
    sem_ie = sems[0]
    sem_ic = sems[1]
    sem_ge = sems[2 + 0 * NSLOT:2 + 1 * NSLOT]
    sem_gc = sems[2 + 1 * NSLOT:2 + 2 * NSLOT]
    sem_s = sems[2 + 2 * NSLOT:2 + 3 * NSLOT]

    wid = lax.axis_index("s") * NC + lax.axis_index("c")
    base = wid * RPW
    lane = lax.iota(jnp.int32, LANES)

    def fix_idx(i):
        off = base + i * SUB
        loc = i * SUB

        def fix(k, c):
            sl = pl.ds(loc + k * LANES, LANES)
            s_v = lax.rem(off + k * LANES + lane, S)
            idx_c[sl] = idx_c[sl] * S + s_v
            return c

        lax.fori_loop(0, SUB // LANES, fix, 0)

    def fire_ge(i, s):
        pltpu.async_copy(emb_hbm.at[idx_e.at[pl.ds(i * SUB, SUB)]],
                         rows_e.at[s], sem_ge[s])

    def wait_ge(i, s):
        pltpu.make_async_copy(emb_hbm.at[idx_e.at[pl.ds(i * SUB, SUB)]],
                              rows_e.at[s], sem_ge[s]).wait()

    def fire_gc(i, s):
        pltpu.async_copy(combo_sh.at[idx_c.at[pl.ds(i * SUB, SUB)]],
                         rows_e.at[s], sem_gc[s], add=True)

    def wait_gc(i, s):
        pltpu.make_async_copy(combo_sh.at[idx_c.at[pl.ds(i * SUB, SUB)]],
                              rows_e.at[s], sem_gc[s]).wait()

    def fire_scatter(i, s):
        off = base + i * SUB
        pltpu.async_copy(rows_e.at[s], out_hbm.at[pl.ds(off, SUB)], sem_s[s])

    def wait_scatter(i, s):
        off = base + i * SUB
        pltpu.make_async_copy(rows_e.at[s], out_hbm.at[pl.ds(off, SUB)],
                              sem_s[s]).wait()

    idx_e_cp = pltpu.async_copy(char_hbm.at[pl.ds(base, RPW)], idx_e, sem_ie)
    idx_c_cp = pltpu.async_copy(pad_hbm.at[pl.ds(base, RPW)], idx_c, sem_ic)

    stage_rows = 40
    stage_start = lax.axis_index("s") * stage_rows
    stage_start = jnp.minimum(stage_start, 3 * S - stage_rows)
    p0 = stage_start // S
    s0 = stage_start - p0 * S
    pltpu.sync_copy(pos_hbm.at[pl.ds(s0, stage_rows)], pos_v)
    pltpu.sync_copy(padtab_hbm, pad_v)

    def combo_row(r, c):
        for d in range(DIM // LANES):
            sl = pl.ds(d * LANES, LANES)
            rows_e[0, r, sl] = pos_v[r, sl] + pad_v[p0, sl]
        return c

    lax.fori_loop(0, stage_rows, combo_row, 0)
    pltpu.sync_copy(rows_e.at[0, pl.ds(0, stage_rows)],
                    combo_sh.at[pl.ds(stage_start, stage_rows)])

    plsc.subcore_barrier()

    idx_e_cp.wait()
    idx_c_cp.wait()

    fix_idx(0)
    fire_ge(0, 0)

    def body(t, carry):
        for j in range(NSLOT):
            i = NSLOT * t + j
            s_0 = j
            s_c = (j - 1) % NSLOT
            s_g = (j + 1) % NSLOT

            @pl.when(jnp.logical_and(i >= 1, i <= NCH))
            def _():
                wait_gc(i - 1, s_c)
                fire_scatter(i - 1, s_c)

            @pl.when(jnp.logical_and(i >= NSLOT - 1, i <= NCH + NSLOT - 2))
            def _():
                wait_scatter(i + 1 - NSLOT, s_g)

            @pl.when(i <= NCH - 2)
            def _():
                fix_idx(i + 1)
                fire_ge(i + 1, s_g)

            @pl.when(i <= NCH - 1)
            def _():
                wait_ge(i, s_0)
                fire_gc(i, s_0)

        return carry

    n_iter = (NCH + NSLOT + NSLOT - 1) // NSLOT
    lax.fori_loop(0, n_iter, body, 0)


_sc_lookup = functools.partial(
    pl.kernel,
    mesh=plsc.VectorSubcoreMesh(core_axis_name="c", subcore_axis_name="s"),
    out_type=jax.ShapeDtypeStruct((N, DIM), jnp.float32),
    scratch_types=[
        pltpu.VMEM((RPW,), jnp.int32),
        pltpu.VMEM((RPW,), jnp.int32),
        pltpu.VMEM((NSLOT, SUB, DIM), jnp.float32),
        pltpu.VMEM((40, DIM), jnp.float32),
        pltpu.VMEM((3, DIM), jnp.float32),
        pltpu.VMEM_SHARED((3 * S, DIM), jnp.float32),
    ] + [pltpu.SemaphoreType.DMA] * (2 + 3 * NSLOT),
)(_sc_body)


@jax.jit
def kernel(char_ids, pad_ids, embedding, pos_embedding, padding_embedding):
    char_flat = char_ids.reshape(N).astype(jnp.int32)
    pad_flat = pad_ids.reshape(N).astype(jnp.int32)
    out = _sc_lookup(char_flat, pad_flat, embedding, pos_embedding,
                     padding_embedding)
    return out.reshape(B, S, DIM)
